# Initial kernel scaffold; baseline (speedup 1.0000x reference)
import jax
import jax.numpy as jnp
from jax.experimental import pallas as pl


def kernel(inputs, embedding):
    raise NotImplementedError("write your pallas kernel here")


# speedup vs baseline: 1.3511x; 1.3511x over previous
import functools

import jax
import jax.numpy as jnp
from jax import lax
from jax.experimental import pallas as pl
from jax.experimental.pallas import tpu as pltpu
from jax.experimental.pallas import tpu_sc as plsc

M_TOTAL = 18432
N_CODES = 8192
D = 64
COMMIT = 0.25

M_BLK = 512
N_WINDOWS = (0, 4096, 8192)
N_BLK = 1024

NC = 2
NS = 16
NW = NC * NS
BPW = M_TOTAL // NW
CH = 96
NCH = BPW // CH


def _argmin_body(x_ref, e_ref, a_ref, b_ref, idx_ref):
    x = x_ref[...]
    a = a_ref[...]
    best_val = jnp.full((M_BLK,), jnp.inf, jnp.float32)
    best_idx = jnp.zeros((M_BLK,), jnp.int32)
    for s, t in zip(N_WINDOWS[:-1], N_WINDOWS[1:]):
        wmin = jnp.full((M_BLK,), jnp.inf, jnp.float32)
        widx = jnp.zeros((M_BLK,), jnp.int32)
        for u in range(s, t, N_BLK):
            e = e_ref[pl.ds(u, N_BLK), :]
            b = b_ref[pl.ds(u, N_BLK)]
            c = lax.dot_general(x, e, (((1,), (1,)), ((), ())),
                                preferred_element_type=jnp.float32)
            dist = (a[:, None] + b[None, :]) - 2.0 * c
            cmin = jnp.min(dist, axis=1)
            ids = lax.broadcasted_iota(jnp.int32, (M_BLK, N_BLK), 1)
            loc = jnp.min(jnp.where(dist == cmin[:, None], ids,
                                    jnp.int32(2**30)), axis=1) + jnp.int32(u)
            take = cmin < wmin
            wmin = jnp.minimum(cmin, wmin)
            widx = jnp.where(take, loc, widx)
        take = wmin < best_val
        best_val = jnp.minimum(wmin, best_val)
        best_val = best_val.astype(jnp.bfloat16).astype(jnp.float32)
        best_idx = jnp.where(take, widx, best_idx)
    idx_ref[...] = best_idx


def _argmin_call(inputs, embedding):
    a = jnp.sum(inputs ** 2, axis=1, keepdims=True)[:, 0]
    b = jnp.sum(embedding ** 2, axis=1)
    return pl.pallas_call(
        _argmin_body,
        grid=(M_TOTAL // M_BLK,),
        in_specs=[
            pl.BlockSpec((M_BLK, D), lambda i: (i, 0)),
            pl.BlockSpec((N_CODES, D), lambda i: (0, 0)),
            pl.BlockSpec((M_BLK,), lambda i: (i,)),
            pl.BlockSpec((N_CODES,), lambda i: (0,)),
        ],
        out_specs=pl.BlockSpec((M_BLK,), lambda i: (i,)),
        out_shape=jax.ShapeDtypeStruct((M_TOTAL,), jnp.int32),
        compiler_params=pltpu.CompilerParams(
            dimension_semantics=("arbitrary",),
        ),
    )(inputs, embedding, a, b)


def _sc_body(idx_hbm, x_hbm, table_hbm,
             q_hbm, counts_hbm, part_hbm,
             idx_v, rows_v, x_v, ones_v, acc_v, zero_v, shared_counts,
             gsem, xsem, ssem):
    cid = lax.axis_index("c")
    sid = lax.axis_index("s")
    wid = sid * NC + cid
    base = wid * BPW

    for c in range(NCH):
        pltpu.sync_copy(idx_hbm.at[pl.ds(base + c * CH, CH)], idx_v.at[c])
    gathers = [pltpu.async_copy(table_hbm.at[idx_v.at[c]],
                                rows_v.at[pl.ds(c * CH, CH)], gsem)
               for c in range(NCH)]
    xcopy = pltpu.async_copy(x_hbm.at[pl.ds(base, BPW)], x_v, xsem)

    ones = jnp.ones((16,), jnp.float32)
    zero = jnp.zeros((16,), jnp.float32)
    for j in range(CH // 16):
        ones_v[pl.ds(j * 16, 16)] = ones

    def _zero_body(j, _):
        zero_v[pl.ds(j * 16, 16)] = zero
        return 0
    lax.fori_loop(0, N_CODES // 16, _zero_body, 0, unroll=8)

    for g in gathers:
        g.wait()
    xcopy.wait()

    @pl.when(sid == 0)
    def _():
        pltpu.sync_copy(zero_v, shared_counts)
    plsc.subcore_barrier()
    for c in range(NCH):
        pltpu.async_copy(ones_v, shared_counts.at[idx_v.at[c]], ssem,
                         add=True).wait()

    def _st_body(r, acc):
        for k in range(D // 16):
            sl = pl.ds(k * 16, 16)
            q = rows_v[r, sl]
            x = x_v[r, sl]
            dlt = q - x
            acc = acc + dlt * dlt
            rows_v[r, sl] = x + dlt
        return acc
    acc = lax.fori_loop(0, BPW, _st_body, jnp.zeros((16,), jnp.float32),
                        unroll=8)
    acc_v[...] = acc

    pltpu.sync_copy(rows_v, q_hbm.at[pl.ds(base, BPW)])
    pltpu.sync_copy(acc_v, part_hbm.at[wid])

    plsc.subcore_barrier()

    @pl.when(sid == 0)
    def _():
        pltpu.sync_copy(shared_counts, counts_hbm.at[cid])


def _sc_call(idx, inputs, embedding):
    mesh = plsc.VectorSubcoreMesh(core_axis_name="c", subcore_axis_name="s",
                                  num_cores=NC, num_subcores=NS)
    f = pl.kernel(
        _sc_body,
        out_type=[
            jax.ShapeDtypeStruct((M_TOTAL, D), jnp.float32),
            jax.ShapeDtypeStruct((NC, N_CODES), jnp.float32),
            jax.ShapeDtypeStruct((NW, 16), jnp.float32),
        ],
        mesh=mesh,
        scratch_types=[
            pltpu.VMEM((NCH, CH), jnp.int32),
            pltpu.VMEM((BPW, D), jnp.float32),
            pltpu.VMEM((BPW, D), jnp.float32),
            pltpu.VMEM((CH,), jnp.float32),
            pltpu.VMEM((16,), jnp.float32),
            pltpu.VMEM((N_CODES,), jnp.float32),
            pltpu.VMEM_SHARED((N_CODES,), jnp.float32),
            pltpu.SemaphoreType.DMA,
            pltpu.SemaphoreType.DMA,
            pltpu.SemaphoreType.DMA,
        ],
        compiler_params=pltpu.CompilerParams(use_tc_tiling_on_sc=False),
    )
    return f(idx, inputs, embedding)


def _final_body(counts_ref, part_ref, loss_ref, perp_ref):
    counts = counts_ref[0, :] + counts_ref[1, :]
    avg = counts / jnp.float32(M_TOTAL)
    ent = jnp.sum(avg * jnp.log(avg + 1e-10))
    perp_ref[...] = jnp.exp(jnp.full((8, 128), -ent, jnp.float32))
    total = jnp.sum(part_ref[...])
    loss_ref[...] = jnp.full((8, 128),
                             COMMIT * (total / jnp.float32(M_TOTAL * D)),
                             jnp.float32)


def _final_call(counts, partials):
    return pl.pallas_call(
        _final_body,
        out_shape=[
            jax.ShapeDtypeStruct((8, 128), jnp.float32),
            jax.ShapeDtypeStruct((8, 128), jnp.float32),
        ],
    )(counts, partials)


def kernel(inputs, embedding):
    idx = _argmin_call(inputs, embedding)
    quantized_st, counts, partials = _sc_call(idx, inputs, embedding)
    loss_b, perp_b = _final_call(counts, partials)
    return quantized_st, loss_b[0, 0], perp_b[0, 0]


# speedup vs baseline: 1.4082x; 1.0423x over previous
import functools

import jax
import jax.numpy as jnp
from jax import lax
from jax.experimental import pallas as pl
from jax.experimental.pallas import tpu as pltpu
from jax.experimental.pallas import tpu_sc as plsc

M_TOTAL = 18432
N_CODES = 8192
D = 64
COMMIT = 0.25

M_BLK = 1024
N_WINDOWS = (0, 4096, 8192)
N_BLK = 1024

NC = 2
NS = 16
NW = NC * NS
BPW = M_TOTAL // NW
CH = 96
NCH = BPW // CH


def _argmin_body(x_ref, e_ref, a_ref, b_ref, idx_ref):
    x = x_ref[...]
    a = a_ref[...]
    best_val = jnp.full((M_BLK,), jnp.inf, jnp.float32)
    best_idx = jnp.zeros((M_BLK,), jnp.int32)
    for s, t in zip(N_WINDOWS[:-1], N_WINDOWS[1:]):
        wmin = jnp.full((M_BLK,), jnp.inf, jnp.float32)
        widx = jnp.zeros((M_BLK,), jnp.int32)
        for u in range(s, t, N_BLK):
            e = e_ref[pl.ds(u, N_BLK), :]
            b = b_ref[pl.ds(u, N_BLK)]
            c = lax.dot_general(x, e, (((1,), (1,)), ((), ())),
                                preferred_element_type=jnp.float32)
            dist = (a[:, None] + b[None, :]) - 2.0 * c
            cmin = jnp.min(dist, axis=1)
            ids = lax.broadcasted_iota(jnp.int32, (M_BLK, N_BLK), 1)
            loc = jnp.min(jnp.where(dist == cmin[:, None], ids,
                                    jnp.int32(2**30)), axis=1) + jnp.int32(u)
            take = cmin < wmin
            wmin = jnp.minimum(cmin, wmin)
            widx = jnp.where(take, loc, widx)
        take = wmin < best_val
        best_val = jnp.minimum(wmin, best_val)
        best_val = best_val.astype(jnp.bfloat16).astype(jnp.float32)
        best_idx = jnp.where(take, widx, best_idx)
    idx_ref[...] = best_idx


def _argmin_call(inputs, embedding):
    a = jnp.sum(inputs ** 2, axis=1, keepdims=True)[:, 0]
    b = jnp.sum(embedding ** 2, axis=1)
    return pl.pallas_call(
        _argmin_body,
        grid=(M_TOTAL // M_BLK,),
        in_specs=[
            pl.BlockSpec((M_BLK, D), lambda i: (i, 0)),
            pl.BlockSpec((N_CODES, D), lambda i: (0, 0)),
            pl.BlockSpec((M_BLK,), lambda i: (i,)),
            pl.BlockSpec((N_CODES,), lambda i: (0,)),
        ],
        out_specs=pl.BlockSpec((M_BLK,), lambda i: (i,)),
        out_shape=jax.ShapeDtypeStruct((M_TOTAL,), jnp.int32),
        compiler_params=pltpu.CompilerParams(
            dimension_semantics=("arbitrary",),
        ),
    )(inputs, embedding, a, b)


def _sc_body(idx_hbm, x_hbm, table_hbm,
             q_hbm, counts_hbm, part_hbm,
             idx_v, rows_v, x_v, ones_v, acc_v, zero_v, shared_counts,
             gsem, xsem, ssem):
    cid = lax.axis_index("c")
    sid = lax.axis_index("s")
    wid = sid * NC + cid
    base = wid * BPW

    for c in range(NCH):
        pltpu.sync_copy(idx_hbm.at[pl.ds(base + c * CH, CH)], idx_v.at[c])
    gathers = [pltpu.async_copy(table_hbm.at[idx_v.at[c]],
                                rows_v.at[pl.ds(c * CH, CH)], gsem)
               for c in range(NCH)]
    xcopy = pltpu.async_copy(x_hbm.at[pl.ds(base, BPW)], x_v, xsem)

    ones = jnp.ones((16,), jnp.float32)
    zero = jnp.zeros((16,), jnp.float32)
    for j in range(CH // 16):
        ones_v[pl.ds(j * 16, 16)] = ones

    def _zero_body(j, _):
        zero_v[pl.ds(j * 16, 16)] = zero
        return 0
    lax.fori_loop(0, N_CODES // 16, _zero_body, 0, unroll=8)

    for g in gathers:
        g.wait()
    xcopy.wait()

    @pl.when(sid == 0)
    def _():
        pltpu.sync_copy(zero_v, shared_counts)
    plsc.subcore_barrier()
    for c in range(NCH):
        pltpu.async_copy(ones_v, shared_counts.at[idx_v.at[c]], ssem,
                         add=True).wait()

    def _st_body(r, acc):
        for k in range(D // 16):
            sl = pl.ds(k * 16, 16)
            q = rows_v[r, sl]
            x = x_v[r, sl]
            dlt = q - x
            acc = acc + dlt * dlt
            rows_v[r, sl] = x + dlt
        return acc
    acc = lax.fori_loop(0, BPW, _st_body, jnp.zeros((16,), jnp.float32),
                        unroll=8)
    acc_v[...] = acc

    pltpu.sync_copy(rows_v, q_hbm.at[pl.ds(base, BPW)])
    pltpu.sync_copy(acc_v, part_hbm.at[wid])

    plsc.subcore_barrier()

    @pl.when(sid == 0)
    def _():
        pltpu.sync_copy(shared_counts, counts_hbm.at[cid])


def _sc_call(idx, inputs, embedding):
    mesh = plsc.VectorSubcoreMesh(core_axis_name="c", subcore_axis_name="s",
                                  num_cores=NC, num_subcores=NS)
    f = pl.kernel(
        _sc_body,
        out_type=[
            jax.ShapeDtypeStruct((M_TOTAL, D), jnp.float32),
            jax.ShapeDtypeStruct((NC, N_CODES), jnp.float32),
            jax.ShapeDtypeStruct((NW, 16), jnp.float32),
        ],
        mesh=mesh,
        scratch_types=[
            pltpu.VMEM((NCH, CH), jnp.int32),
            pltpu.VMEM((BPW, D), jnp.float32),
            pltpu.VMEM((BPW, D), jnp.float32),
            pltpu.VMEM((CH,), jnp.float32),
            pltpu.VMEM((16,), jnp.float32),
            pltpu.VMEM((N_CODES,), jnp.float32),
            pltpu.VMEM_SHARED((N_CODES,), jnp.float32),
            pltpu.SemaphoreType.DMA,
            pltpu.SemaphoreType.DMA,
            pltpu.SemaphoreType.DMA,
        ],
        compiler_params=pltpu.CompilerParams(use_tc_tiling_on_sc=False),
    )
    return f(idx, inputs, embedding)


def _final_body(counts_ref, part_ref, loss_ref, perp_ref):
    counts = counts_ref[0, :] + counts_ref[1, :]
    avg = counts / jnp.float32(M_TOTAL)
    ent = jnp.sum(avg * jnp.log(avg + 1e-10))
    perp_ref[...] = jnp.exp(jnp.full((8, 128), -ent, jnp.float32))
    total = jnp.sum(part_ref[...])
    loss_ref[...] = jnp.full((8, 128),
                             COMMIT * (total / jnp.float32(M_TOTAL * D)),
                             jnp.float32)


def _final_call(counts, partials):
    return pl.pallas_call(
        _final_body,
        out_shape=[
            jax.ShapeDtypeStruct((8, 128), jnp.float32),
            jax.ShapeDtypeStruct((8, 128), jnp.float32),
        ],
    )(counts, partials)


def kernel(inputs, embedding):
    idx = _argmin_call(inputs, embedding)
    quantized_st, counts, partials = _sc_call(idx, inputs, embedding)
    loss_b, perp_b = _final_call(counts, partials)
    return quantized_st, loss_b[0, 0], perp_b[0, 0]


# speedup vs baseline: 1.5964x; 1.1336x over previous
import functools

import jax
import jax.numpy as jnp
from jax import lax
from jax.experimental import pallas as pl
from jax.experimental.pallas import tpu as pltpu
from jax.experimental.pallas import tpu_sc as plsc

M_TOTAL = 18432
N_CODES = 8192
D = 64
COMMIT = 0.25

M_BLK = 1024
N_WINDOWS = (0, 4096, 8192)
N_BLK = 1024

NC = 2
NS = 16
NW = NC * NS
BPW = M_TOTAL // NW
CH = 96
NCH = BPW // CH


def _argmin_body(x_ref, e_ref, a_ref, b_ref, idx_ref):
    x = x_ref[...]
    a = a_ref[...]
    best_val = jnp.full((M_BLK,), jnp.inf, jnp.float32)
    best_idx = jnp.zeros((M_BLK,), jnp.int32)
    for s, t in zip(N_WINDOWS[:-1], N_WINDOWS[1:]):
        wmin = jnp.full((M_BLK,), jnp.inf, jnp.float32)
        widx = jnp.zeros((M_BLK,), jnp.int32)
        for u in range(s, t, N_BLK):
            e = e_ref[pl.ds(u, N_BLK), :]
            b = b_ref[pl.ds(u, N_BLK)]
            c = lax.dot_general(x, e, (((1,), (1,)), ((), ())),
                                preferred_element_type=jnp.float32)
            dist = (a[:, None] + b[None, :]) - 2.0 * c
            cmin = jnp.min(dist, axis=1)
            ids = lax.broadcasted_iota(jnp.int32, (M_BLK, N_BLK),
                                       1).astype(jnp.float32)
            locf = jnp.min(jnp.where(dist == cmin[:, None], ids,
                                     jnp.float32(1e30)), axis=1)
            loc = locf.astype(jnp.int32) + jnp.int32(u)
            take = cmin < wmin
            wmin = jnp.minimum(cmin, wmin)
            widx = jnp.where(take, loc, widx)
        take = wmin < best_val
        best_val = jnp.minimum(wmin, best_val)
        best_val = best_val.astype(jnp.bfloat16).astype(jnp.float32)
        best_idx = jnp.where(take, widx, best_idx)
    idx_ref[...] = best_idx


def _argmin_call(inputs, embedding):
    a = jnp.sum(inputs ** 2, axis=1, keepdims=True)[:, 0]
    b = jnp.sum(embedding ** 2, axis=1)
    return pl.pallas_call(
        _argmin_body,
        grid=(M_TOTAL // M_BLK,),
        in_specs=[
            pl.BlockSpec((M_BLK, D), lambda i: (i, 0)),
            pl.BlockSpec((N_CODES, D), lambda i: (0, 0)),
            pl.BlockSpec((M_BLK,), lambda i: (i,)),
            pl.BlockSpec((N_CODES,), lambda i: (0,)),
        ],
        out_specs=pl.BlockSpec((M_BLK,), lambda i: (i,)),
        out_shape=jax.ShapeDtypeStruct((M_TOTAL,), jnp.int32),
        compiler_params=pltpu.CompilerParams(
            dimension_semantics=("arbitrary",),
        ),
    )(inputs, embedding, a, b)


def _sc_body(idx_hbm, x_hbm, table_hbm,
             q_hbm, counts_hbm, part_hbm,
             idx_v, rows_v, x_v, ones_v, acc_v, zero_v, shared_counts,
             gsem, xsem, ssem):
    cid = lax.axis_index("c")
    sid = lax.axis_index("s")
    wid = sid * NC + cid
    base = wid * BPW

    for c in range(NCH):
        pltpu.sync_copy(idx_hbm.at[pl.ds(base + c * CH, CH)], idx_v.at[c])
    gathers = [pltpu.async_copy(table_hbm.at[idx_v.at[c]],
                                rows_v.at[pl.ds(c * CH, CH)], gsem)
               for c in range(NCH)]
    xcopy = pltpu.async_copy(x_hbm.at[pl.ds(base, BPW)], x_v, xsem)

    ones = jnp.ones((16,), jnp.float32)
    zero = jnp.zeros((16,), jnp.float32)
    for j in range(CH // 16):
        ones_v[pl.ds(j * 16, 16)] = ones

    def _zero_body(j, _):
        zero_v[pl.ds(j * 16, 16)] = zero
        return 0
    lax.fori_loop(0, N_CODES // 16, _zero_body, 0, unroll=8)

    for g in gathers:
        g.wait()
    xcopy.wait()

    @pl.when(sid == 0)
    def _():
        pltpu.sync_copy(zero_v, shared_counts)
    plsc.subcore_barrier()
    for c in range(NCH):
        pltpu.async_copy(ones_v, shared_counts.at[idx_v.at[c]], ssem,
                         add=True).wait()

    def _st_body(r, acc):
        for k in range(D // 16):
            sl = pl.ds(k * 16, 16)
            q = rows_v[r, sl]
            x = x_v[r, sl]
            dlt = q - x
            acc = acc + dlt * dlt
            rows_v[r, sl] = x + dlt
        return acc
    acc = lax.fori_loop(0, BPW, _st_body, jnp.zeros((16,), jnp.float32),
                        unroll=8)
    acc_v[...] = acc

    pltpu.sync_copy(rows_v, q_hbm.at[pl.ds(base, BPW)])
    pltpu.sync_copy(acc_v, part_hbm.at[wid])

    plsc.subcore_barrier()

    @pl.when(sid == 0)
    def _():
        pltpu.sync_copy(shared_counts, counts_hbm.at[cid])


def _sc_call(idx, inputs, embedding):
    mesh = plsc.VectorSubcoreMesh(core_axis_name="c", subcore_axis_name="s",
                                  num_cores=NC, num_subcores=NS)
    f = pl.kernel(
        _sc_body,
        out_type=[
            jax.ShapeDtypeStruct((M_TOTAL, D), jnp.float32),
            jax.ShapeDtypeStruct((NC, N_CODES), jnp.float32),
            jax.ShapeDtypeStruct((NW, 16), jnp.float32),
        ],
        mesh=mesh,
        scratch_types=[
            pltpu.VMEM((NCH, CH), jnp.int32),
            pltpu.VMEM((BPW, D), jnp.float32),
            pltpu.VMEM((BPW, D), jnp.float32),
            pltpu.VMEM((CH,), jnp.float32),
            pltpu.VMEM((16,), jnp.float32),
            pltpu.VMEM((N_CODES,), jnp.float32),
            pltpu.VMEM_SHARED((N_CODES,), jnp.float32),
            pltpu.SemaphoreType.DMA,
            pltpu.SemaphoreType.DMA,
            pltpu.SemaphoreType.DMA,
        ],
        compiler_params=pltpu.CompilerParams(use_tc_tiling_on_sc=False),
    )
    return f(idx, inputs, embedding)


def _final_body(counts_ref, part_ref, loss_ref, perp_ref):
    counts = counts_ref[0, :] + counts_ref[1, :]
    avg = counts / jnp.float32(M_TOTAL)
    ent = jnp.sum(avg * jnp.log(avg + 1e-10))
    perp_ref[...] = jnp.exp(jnp.full((8, 128), -ent, jnp.float32))
    total = jnp.sum(part_ref[...])
    loss_ref[...] = jnp.full((8, 128),
                             COMMIT * (total / jnp.float32(M_TOTAL * D)),
                             jnp.float32)


def _final_call(counts, partials):
    return pl.pallas_call(
        _final_body,
        out_shape=[
            jax.ShapeDtypeStruct((8, 128), jnp.float32),
            jax.ShapeDtypeStruct((8, 128), jnp.float32),
        ],
    )(counts, partials)


def kernel(inputs, embedding):
    idx = _argmin_call(inputs, embedding)
    quantized_st, counts, partials = _sc_call(idx, inputs, embedding)
    loss_b, perp_b = _final_call(counts, partials)
    return quantized_st, loss_b[0, 0], perp_b[0, 0]


# speedup vs baseline: 1.6424x; 1.0288x over previous
import functools

import jax
import jax.numpy as jnp
from jax import lax
from jax.experimental import pallas as pl
from jax.experimental.pallas import tpu as pltpu
from jax.experimental.pallas import tpu_sc as plsc

M_TOTAL = 18432
N_CODES = 8192
D = 64
COMMIT = 0.25

M_BLK = 1024
N_WINDOWS = (0, 4096, 8192)
N_BLK = 2048

NC = 2
NS = 16
NW = NC * NS
BPW = M_TOTAL // NW
CH = 96
NCH = BPW // CH


def _argmin_body(x_ref, e_ref, a_ref, b_ref, idx_ref):
    x = x_ref[...]
    a = a_ref[...]
    best_val = jnp.full((M_BLK,), jnp.inf, jnp.float32)
    best_idx = jnp.zeros((M_BLK,), jnp.int32)
    for s, t in zip(N_WINDOWS[:-1], N_WINDOWS[1:]):
        wmin = jnp.full((M_BLK,), jnp.inf, jnp.float32)
        widx = jnp.zeros((M_BLK,), jnp.int32)
        for u in range(s, t, N_BLK):
            e = e_ref[pl.ds(u, N_BLK), :]
            b = b_ref[pl.ds(u, N_BLK)]
            c = lax.dot_general(x, e, (((1,), (1,)), ((), ())),
                                preferred_element_type=jnp.float32)
            dist = (a[:, None] + b[None, :]) - 2.0 * c
            cmin = jnp.min(dist, axis=1)
            ids = lax.broadcasted_iota(jnp.int32, (M_BLK, N_BLK),
                                       1).astype(jnp.float32)
            locf = jnp.min(jnp.where(dist == cmin[:, None], ids,
                                     jnp.float32(1e30)), axis=1)
            loc = locf.astype(jnp.int32) + jnp.int32(u)
            take = cmin < wmin
            wmin = jnp.minimum(cmin, wmin)
            widx = jnp.where(take, loc, widx)
        take = wmin < best_val
        best_val = jnp.minimum(wmin, best_val)
        best_val = best_val.astype(jnp.bfloat16).astype(jnp.float32)
        best_idx = jnp.where(take, widx, best_idx)
    idx_ref[...] = best_idx


def _argmin_call(inputs, embedding):
    a = jnp.sum(inputs ** 2, axis=1, keepdims=True)[:, 0]
    b = jnp.sum(embedding ** 2, axis=1)
    return pl.pallas_call(
        _argmin_body,
        grid=(M_TOTAL // M_BLK,),
        in_specs=[
            pl.BlockSpec((M_BLK, D), lambda i: (i, 0)),
            pl.BlockSpec((N_CODES, D), lambda i: (0, 0)),
            pl.BlockSpec((M_BLK,), lambda i: (i,)),
            pl.BlockSpec((N_CODES,), lambda i: (0,)),
        ],
        out_specs=pl.BlockSpec((M_BLK,), lambda i: (i,)),
        out_shape=jax.ShapeDtypeStruct((M_TOTAL,), jnp.int32),
        compiler_params=pltpu.CompilerParams(
            dimension_semantics=("arbitrary",),
        ),
    )(inputs, embedding, a, b)


def _sc_body(idx_hbm, x_hbm, table_hbm,
             q_hbm, counts_hbm, part_hbm,
             idx_v, rows_v, x_v, ones_v, acc_v, zero_v, shared_counts,
             gsem, xsem, ssem):
    cid = lax.axis_index("c")
    sid = lax.axis_index("s")
    wid = sid * NC + cid
    base = wid * BPW

    for c in range(NCH):
        pltpu.sync_copy(idx_hbm.at[pl.ds(base + c * CH, CH)], idx_v.at[c])
    gathers = [pltpu.async_copy(table_hbm.at[idx_v.at[c]],
                                rows_v.at[pl.ds(c * CH, CH)], gsem)
               for c in range(NCH)]
    xcopy = pltpu.async_copy(x_hbm.at[pl.ds(base, BPW)], x_v, xsem)

    ones = jnp.ones((16,), jnp.float32)
    zero = jnp.zeros((16,), jnp.float32)
    for j in range(CH // 16):
        ones_v[pl.ds(j * 16, 16)] = ones

    def _zero_body(j, _):
        zero_v[pl.ds(j * 16, 16)] = zero
        return 0
    lax.fori_loop(0, N_CODES // 16, _zero_body, 0, unroll=8)

    for g in gathers:
        g.wait()
    xcopy.wait()

    @pl.when(sid == 0)
    def _():
        pltpu.sync_copy(zero_v, shared_counts)
    plsc.subcore_barrier()
    for c in range(NCH):
        pltpu.async_copy(ones_v, shared_counts.at[idx_v.at[c]], ssem,
                         add=True).wait()

    def _st_body(r, acc):
        for k in range(D // 16):
            sl = pl.ds(k * 16, 16)
            q = rows_v[r, sl]
            x = x_v[r, sl]
            dlt = q - x
            acc = acc + dlt * dlt
            rows_v[r, sl] = x + dlt
        return acc
    acc = lax.fori_loop(0, BPW, _st_body, jnp.zeros((16,), jnp.float32),
                        unroll=8)
    acc_v[...] = acc

    pltpu.sync_copy(rows_v, q_hbm.at[pl.ds(base, BPW)])
    pltpu.sync_copy(acc_v, part_hbm.at[wid])

    plsc.subcore_barrier()

    @pl.when(sid == 0)
    def _():
        pltpu.sync_copy(shared_counts, counts_hbm.at[cid])


def _sc_call(idx, inputs, embedding):
    mesh = plsc.VectorSubcoreMesh(core_axis_name="c", subcore_axis_name="s",
                                  num_cores=NC, num_subcores=NS)
    f = pl.kernel(
        _sc_body,
        out_type=[
            jax.ShapeDtypeStruct((M_TOTAL, D), jnp.float32),
            jax.ShapeDtypeStruct((NC, N_CODES), jnp.float32),
            jax.ShapeDtypeStruct((NW, 16), jnp.float32),
        ],
        mesh=mesh,
        scratch_types=[
            pltpu.VMEM((NCH, CH), jnp.int32),
            pltpu.VMEM((BPW, D), jnp.float32),
            pltpu.VMEM((BPW, D), jnp.float32),
            pltpu.VMEM((CH,), jnp.float32),
            pltpu.VMEM((16,), jnp.float32),
            pltpu.VMEM((N_CODES,), jnp.float32),
            pltpu.VMEM_SHARED((N_CODES,), jnp.float32),
            pltpu.SemaphoreType.DMA,
            pltpu.SemaphoreType.DMA,
            pltpu.SemaphoreType.DMA,
        ],
        compiler_params=pltpu.CompilerParams(use_tc_tiling_on_sc=False),
    )
    return f(idx, inputs, embedding)


def _final_body(counts_ref, part_ref, loss_ref, perp_ref):
    counts = counts_ref[0, :] + counts_ref[1, :]
    avg = counts / jnp.float32(M_TOTAL)
    ent = jnp.sum(avg * jnp.log(avg + 1e-10))
    perp_ref[...] = jnp.exp(jnp.full((8, 128), -ent, jnp.float32))
    total = jnp.sum(part_ref[...])
    loss_ref[...] = jnp.full((8, 128),
                             COMMIT * (total / jnp.float32(M_TOTAL * D)),
                             jnp.float32)


def _final_call(counts, partials):
    return pl.pallas_call(
        _final_body,
        out_shape=[
            jax.ShapeDtypeStruct((8, 128), jnp.float32),
            jax.ShapeDtypeStruct((8, 128), jnp.float32),
        ],
    )(counts, partials)


def kernel(inputs, embedding):
    idx = _argmin_call(inputs, embedding)
    quantized_st, counts, partials = _sc_call(idx, inputs, embedding)
    loss_b, perp_b = _final_call(counts, partials)
    return quantized_st, loss_b[0, 0], perp_b[0, 0]


# speedup vs baseline: 1.6470x; 1.0028x over previous
import functools

import jax
import jax.numpy as jnp
from jax import lax
from jax.experimental import pallas as pl
from jax.experimental.pallas import tpu as pltpu
from jax.experimental.pallas import tpu_sc as plsc

M_TOTAL = 18432
N_CODES = 8192
D = 64
COMMIT = 0.25

M_BLK = 2048
N_WINDOWS = (0, 4096, 8192)
N_BLK = 2048

NC = 2
NS = 16
NW = NC * NS
BPW = M_TOTAL // NW
CH = 96
NCH = BPW // CH


def _argmin_body(x_ref, e_ref, a_ref, b_ref, idx_ref):
    x = x_ref[...]
    a = a_ref[...]
    best_val = jnp.full((M_BLK,), jnp.inf, jnp.float32)
    best_idx = jnp.zeros((M_BLK,), jnp.int32)
    for s, t in zip(N_WINDOWS[:-1], N_WINDOWS[1:]):
        wmin = jnp.full((M_BLK,), jnp.inf, jnp.float32)
        widx = jnp.zeros((M_BLK,), jnp.int32)
        for u in range(s, t, N_BLK):
            e = e_ref[pl.ds(u, N_BLK), :]
            b = b_ref[pl.ds(u, N_BLK)]
            c = lax.dot_general(x, e, (((1,), (1,)), ((), ())),
                                preferred_element_type=jnp.float32)
            dist = (a[:, None] + b[None, :]) - 2.0 * c
            cmin = jnp.min(dist, axis=1)
            ids = lax.broadcasted_iota(jnp.int32, (M_BLK, N_BLK),
                                       1).astype(jnp.float32)
            locf = jnp.min(jnp.where(dist == cmin[:, None], ids,
                                     jnp.float32(1e30)), axis=1)
            loc = locf.astype(jnp.int32) + jnp.int32(u)
            take = cmin < wmin
            wmin = jnp.minimum(cmin, wmin)
            widx = jnp.where(take, loc, widx)
        take = wmin < best_val
        best_val = jnp.minimum(wmin, best_val)
        best_val = best_val.astype(jnp.bfloat16).astype(jnp.float32)
        best_idx = jnp.where(take, widx, best_idx)
    idx_ref[...] = best_idx


def _argmin_call(inputs, embedding):
    a = jnp.sum(inputs ** 2, axis=1, keepdims=True)[:, 0]
    b = jnp.sum(embedding ** 2, axis=1)
    return pl.pallas_call(
        _argmin_body,
        grid=(M_TOTAL // M_BLK,),
        in_specs=[
            pl.BlockSpec((M_BLK, D), lambda i: (i, 0)),
            pl.BlockSpec((N_CODES, D), lambda i: (0, 0)),
            pl.BlockSpec((M_BLK,), lambda i: (i,)),
            pl.BlockSpec((N_CODES,), lambda i: (0,)),
        ],
        out_specs=pl.BlockSpec((M_BLK,), lambda i: (i,)),
        out_shape=jax.ShapeDtypeStruct((M_TOTAL,), jnp.int32),
        compiler_params=pltpu.CompilerParams(
            dimension_semantics=("arbitrary",),
        ),
    )(inputs, embedding, a, b)


def _sc_body(idx_hbm, x_hbm, table_hbm,
             q_hbm, counts_hbm, part_hbm,
             idx_v, rows_v, x_v, ones_v, acc_v, zero_v, shared_counts,
             gsem, xsem, ssem):
    cid = lax.axis_index("c")
    sid = lax.axis_index("s")
    wid = sid * NC + cid
    base = wid * BPW

    for c in range(NCH):
        pltpu.sync_copy(idx_hbm.at[pl.ds(base + c * CH, CH)], idx_v.at[c])
    gathers = [pltpu.async_copy(table_hbm.at[idx_v.at[c]],
                                rows_v.at[pl.ds(c * CH, CH)], gsem)
               for c in range(NCH)]
    xcopy = pltpu.async_copy(x_hbm.at[pl.ds(base, BPW)], x_v, xsem)

    ones = jnp.ones((16,), jnp.float32)
    zero = jnp.zeros((16,), jnp.float32)
    for j in range(CH // 16):
        ones_v[pl.ds(j * 16, 16)] = ones

    def _zero_body(j, _):
        zero_v[pl.ds(j * 16, 16)] = zero
        return 0
    lax.fori_loop(0, N_CODES // 16, _zero_body, 0, unroll=8)

    for g in gathers:
        g.wait()
    xcopy.wait()

    @pl.when(sid == 0)
    def _():
        pltpu.sync_copy(zero_v, shared_counts)
    plsc.subcore_barrier()
    for c in range(NCH):
        pltpu.async_copy(ones_v, shared_counts.at[idx_v.at[c]], ssem,
                         add=True).wait()

    def _st_body(r, acc):
        for k in range(D // 16):
            sl = pl.ds(k * 16, 16)
            q = rows_v[r, sl]
            x = x_v[r, sl]
            dlt = q - x
            acc = acc + dlt * dlt
            rows_v[r, sl] = x + dlt
        return acc
    acc = lax.fori_loop(0, BPW, _st_body, jnp.zeros((16,), jnp.float32),
                        unroll=8)
    acc_v[...] = acc

    pltpu.sync_copy(rows_v, q_hbm.at[pl.ds(base, BPW)])
    pltpu.sync_copy(acc_v, part_hbm.at[wid])

    plsc.subcore_barrier()

    @pl.when(sid == 0)
    def _():
        pltpu.sync_copy(shared_counts, counts_hbm.at[cid])


def _sc_call(idx, inputs, embedding):
    mesh = plsc.VectorSubcoreMesh(core_axis_name="c", subcore_axis_name="s",
                                  num_cores=NC, num_subcores=NS)
    f = pl.kernel(
        _sc_body,
        out_type=[
            jax.ShapeDtypeStruct((M_TOTAL, D), jnp.float32),
            jax.ShapeDtypeStruct((NC, N_CODES), jnp.float32),
            jax.ShapeDtypeStruct((NW, 16), jnp.float32),
        ],
        mesh=mesh,
        scratch_types=[
            pltpu.VMEM((NCH, CH), jnp.int32),
            pltpu.VMEM((BPW, D), jnp.float32),
            pltpu.VMEM((BPW, D), jnp.float32),
            pltpu.VMEM((CH,), jnp.float32),
            pltpu.VMEM((16,), jnp.float32),
            pltpu.VMEM((N_CODES,), jnp.float32),
            pltpu.VMEM_SHARED((N_CODES,), jnp.float32),
            pltpu.SemaphoreType.DMA,
            pltpu.SemaphoreType.DMA,
            pltpu.SemaphoreType.DMA,
        ],
        compiler_params=pltpu.CompilerParams(use_tc_tiling_on_sc=False),
    )
    return f(idx, inputs, embedding)


def _final_body(counts_ref, part_ref, loss_ref, perp_ref):
    counts = counts_ref[0, :] + counts_ref[1, :]
    avg = counts / jnp.float32(M_TOTAL)
    ent = jnp.sum(avg * jnp.log(avg + 1e-10))
    perp_ref[...] = jnp.exp(jnp.full((8, 128), -ent, jnp.float32))
    total = jnp.sum(part_ref[...])
    loss_ref[...] = jnp.full((8, 128),
                             COMMIT * (total / jnp.float32(M_TOTAL * D)),
                             jnp.float32)


def _final_call(counts, partials):
    return pl.pallas_call(
        _final_body,
        out_shape=[
            jax.ShapeDtypeStruct((8, 128), jnp.float32),
            jax.ShapeDtypeStruct((8, 128), jnp.float32),
        ],
    )(counts, partials)


def kernel(inputs, embedding):
    idx = _argmin_call(inputs, embedding)
    quantized_st, counts, partials = _sc_call(idx, inputs, embedding)
    loss_b, perp_b = _final_call(counts, partials)
    return quantized_st, loss_b[0, 0], perp_b[0, 0]
